# speedup vs baseline: 1.3880x; 1.3880x over previous
import functools

import jax
import jax.numpy as jnp
from jax import lax
from jax.experimental import pallas as pl
from jax.experimental.pallas import tpu as pltpu
from jax.experimental.pallas import tpu_sc as plsc

VOCAB = 100000
EMBED_DIM = 768
BATCH = 4
SEQ = 2048

NUM_CORES = 2
NUM_SUBCORES = 16
NW = NUM_CORES * NUM_SUBCORES
TOTAL = BATCH * SEQ
B_PER_W = TOTAL // NW
CHUNK = 64
NCHUNK = B_PER_W // CHUNK


def _make_gather():
    mesh = plsc.VectorSubcoreMesh(core_axis_name="c", subcore_axis_name="s")

    @functools.partial(
        pl.kernel,
        mesh=mesh,
        out_type=jax.ShapeDtypeStruct((TOTAL, EMBED_DIM), jnp.float32),
        scratch_types=[
            pltpu.VMEM((NCHUNK, CHUNK), jnp.int32),
            pltpu.VMEM((CHUNK, EMBED_DIM), jnp.float32),
            pltpu.SemaphoreType.DMA,
        ],
    )
    def k(ids_hbm, table_hbm, out_hbm, idx_v, rows_v, sem):
        wid = lax.axis_index("s") * NUM_CORES + lax.axis_index("c")
        base = wid * B_PER_W
        pltpu.sync_copy(ids_hbm.at[wid], idx_v)
        for j in range(NCHUNK):
            pltpu.async_copy(table_hbm.at[idx_v.at[j]], rows_v, sem).wait()
            pltpu.sync_copy(rows_v, out_hbm.at[pl.ds(base + j * CHUNK, CHUNK)])

    return k


_gather = _make_gather()


def kernel(encoder_input_ids, encoder_attention_mask, embed_table):
    ids = encoder_input_ids.reshape(NW, NCHUNK, CHUNK).astype(jnp.int32)
    out = _gather(ids, embed_table)
    hidden = out.reshape(BATCH, SEQ, EMBED_DIM)
    return (encoder_input_ids, encoder_attention_mask, hidden)


# speedup vs baseline: 1.4125x; 1.0176x over previous
import functools

import jax
import jax.numpy as jnp
from jax import lax
from jax.experimental import pallas as pl
from jax.experimental.pallas import tpu as pltpu
from jax.experimental.pallas import tpu_sc as plsc

VOCAB = 100000
EMBED_DIM = 768
BATCH = 4
SEQ = 2048

NUM_CORES = 2
NUM_SUBCORES = 16
NW = NUM_CORES * NUM_SUBCORES
TOTAL = BATCH * SEQ
B_PER_W = TOTAL // NW
CHUNK = 64
NCHUNK = B_PER_W // CHUNK


def _make_gather():
    mesh = plsc.VectorSubcoreMesh(core_axis_name="c", subcore_axis_name="s")

    @functools.partial(
        pl.kernel,
        mesh=mesh,
        out_type=jax.ShapeDtypeStruct((TOTAL, EMBED_DIM), jnp.float32),
        scratch_types=[
            pltpu.VMEM((NCHUNK, CHUNK), jnp.int32),
            pltpu.VMEM((CHUNK, EMBED_DIM), jnp.float32),
            pltpu.VMEM((CHUNK, EMBED_DIM), jnp.float32),
            pltpu.SemaphoreType.DMA,
            pltpu.SemaphoreType.DMA,
        ],
    )
    def k(ids_hbm, table_hbm, out_hbm, idx_v, rows0, rows1, gsem, wsem):
        wid = lax.axis_index("s") * NUM_CORES + lax.axis_index("c")
        base = wid * B_PER_W
        pltpu.sync_copy(ids_hbm.at[wid], idx_v)
        bufs = (rows0, rows1)
        g = [None, None]
        w = [None, None]
        g[0] = pltpu.async_copy(table_hbm.at[idx_v.at[0]], bufs[0], gsem)
        for j in range(NCHUNK):
            cur = j % 2
            g[cur].wait()
            if j + 1 < NCHUNK:
                if w[1 - cur] is not None:
                    w[1 - cur].wait()
                g[1 - cur] = pltpu.async_copy(
                    table_hbm.at[idx_v.at[j + 1]], bufs[1 - cur], gsem)
            w[cur] = pltpu.async_copy(
                bufs[cur], out_hbm.at[pl.ds(base + j * CHUNK, CHUNK)], wsem)
        w[0].wait()
        w[1].wait()

    return k


_gather = _make_gather()


def kernel(encoder_input_ids, encoder_attention_mask, embed_table):
    ids = encoder_input_ids.reshape(NW, NCHUNK, CHUNK).astype(jnp.int32)
    out = _gather(ids, embed_table)
    hidden = out.reshape(BATCH, SEQ, EMBED_DIM)
    return (encoder_input_ids, encoder_attention_mask, hidden)


# speedup vs baseline: 1.4836x; 1.0504x over previous
import functools

import jax
import jax.numpy as jnp
from jax import lax
from jax.experimental import pallas as pl
from jax.experimental.pallas import tpu as pltpu
from jax.experimental.pallas import tpu_sc as plsc

VOCAB = 100000
EMBED_DIM = 768
BATCH = 4
SEQ = 2048

NUM_CORES = 2
NUM_SUBCORES = 16
NW = NUM_CORES * NUM_SUBCORES
TOTAL = BATCH * SEQ
B_PER_W = TOTAL // NW
CHUNK = 32
NCHUNK = B_PER_W // CHUNK
NBUF = 5
WDELAY = 2


def _make_gather():
    mesh = plsc.VectorSubcoreMesh(core_axis_name="c", subcore_axis_name="s")

    @functools.partial(
        pl.kernel,
        mesh=mesh,
        out_type=jax.ShapeDtypeStruct((TOTAL, EMBED_DIM), jnp.float32),
        scratch_types=[
            pltpu.VMEM((NCHUNK, CHUNK), jnp.int32),
        ] + [
            pltpu.VMEM((CHUNK, EMBED_DIM), jnp.float32) for _ in range(NBUF)
        ] + [
            pltpu.SemaphoreType.DMA,
            pltpu.SemaphoreType.DMA,
        ],
    )
    def k(ids_hbm, table_hbm, out_hbm, idx_v, *rest):
        bufs = rest[:NBUF]
        gsem, wsem = rest[NBUF], rest[NBUF + 1]
        wid = lax.axis_index("s") * NUM_CORES + lax.axis_index("c")
        base = wid * B_PER_W
        pltpu.sync_copy(ids_hbm.at[wid], idx_v)
        g = [None] * NBUF
        w = [None] * NBUF
        for j in range(min(NBUF, NCHUNK)):
            g[j] = pltpu.async_copy(table_hbm.at[idx_v.at[j]], bufs[j], gsem)
        for j in range(NCHUNK):
            b = j % NBUF
            g[b].wait()
            w[b] = pltpu.async_copy(
                bufs[b], out_hbm.at[pl.ds(base + j * CHUNK, CHUNK)], wsem)
            jd = j - WDELAY
            if jd >= 0 and jd + NBUF < NCHUNK:
                bd = jd % NBUF
                w[bd].wait()
                w[bd] = None
                g[bd] = pltpu.async_copy(
                    table_hbm.at[idx_v.at[jd + NBUF]], bufs[bd], gsem)
        for b in range(NBUF):
            if w[b] is not None:
                w[b].wait()

    return k


_gather = _make_gather()


def kernel(encoder_input_ids, encoder_attention_mask, embed_table):
    ids = encoder_input_ids.reshape(NW, NCHUNK, CHUNK).astype(jnp.int32)
    out = _gather(ids, embed_table)
    hidden = out.reshape(BATCH, SEQ, EMBED_DIM)
    return (encoder_input_ids, encoder_attention_mask, hidden)
